# baseline (speedup 1.0000x reference)
import functools

import jax
import jax.numpy as jnp
from jax import lax
from jax.experimental import pallas as pl
from jax.experimental.pallas import tpu as pltpu
from jax.experimental.pallas import tpu_sc as plsc

EMB = 64
HID = 128
NW = 32
BLK = 128
SLICE = 1568
PAD_P = 16 * SLICE

_f32 = jnp.float32


G_IDX = 15


def _split_blocks(nb):
    n_full = nb // NW
    n_rem = nb - n_full * NW
    assert n_full % G_IDX == 0
    return n_full, n_rem


def _sc_segsum64(table, src2d, dst2d, zrows):
    nb = src2d.shape[0]
    n_full, n_rem = _split_blocks(nb)
    nfb = n_full // G_IDX
    mesh = plsc.VectorSubcoreMesh(core_axis_name="c", subcore_axis_name="s")

    @functools.partial(
        pl.kernel,
        out_type=(jax.ShapeDtypeStruct((PAD_P, EMB), _f32),
                  jax.ShapeDtypeStruct((PAD_P, EMB), _f32)),
        mesh=mesh,
        scratch_types=[
            pltpu.VMEM_SHARED((PAD_P, EMB), _f32),
            pltpu.VMEM((G_IDX, BLK), jnp.int32),
            pltpu.VMEM((G_IDX, BLK), jnp.int32),
            pltpu.VMEM((BLK, EMB), _f32),
            pltpu.VMEM((BLK, EMB), _f32),
            pltpu.SemaphoreType.DMA,
            pltpu.SemaphoreType.DMA,
            pltpu.SemaphoreType.DMA,
        ],
        compiler_params=pltpu.CompilerParams(use_tc_tiling_on_sc=False),
    )
    def k(table_h, src_h, dst_h, z_h, out0_h, out1_h, acc,
          si, di, r0, r1, isem, gsem, ssem):
        c = lax.axis_index("c")
        s = lax.axis_index("s")
        w = s * 2 + c
        pltpu.sync_copy(z_h, acc.at[pl.ds(s * SLICE, SLICE)])
        plsc.subcore_barrier()
        base = w * n_full + jnp.minimum(w, n_rem)
        rbuf = (r0, r1)

        def run_batch(b0):
            cpa = pltpu.async_copy(src_h.at[pl.ds(b0, G_IDX)], si, isem)
            cpb = pltpu.async_copy(dst_h.at[pl.ds(b0, G_IDX)], di, isem)
            cpa.wait()
            cpb.wait()
            gat = [None] * G_IDX
            sca = [None] * G_IDX
            gat[0] = pltpu.async_copy(table_h.at[si.at[0]], rbuf[0], gsem)
            for j in range(G_IDX):
                gat[j].wait()
                if j >= 1:
                    sca[j - 1].wait()
                if j + 1 < G_IDX:
                    gat[j + 1] = pltpu.async_copy(
                        table_h.at[si.at[j + 1]], rbuf[(j + 1) % 2], gsem)
                sca[j] = pltpu.async_copy(
                    rbuf[j % 2], acc.at[di.at[j]], ssem, add=True)
            sca[G_IDX - 1].wait()

        def body(i, carry):
            run_batch(base + i * G_IDX)
            return carry

        lax.fori_loop(0, nfb, body, 0)
        if n_rem:
            @pl.when(w < n_rem)
            def _():
                b = base + n_full
                pltpu.sync_copy(src_h.at[pl.ds(b, 1)], si.at[pl.ds(0, 1)])
                pltpu.sync_copy(dst_h.at[pl.ds(b, 1)], di.at[pl.ds(0, 1)])
                pltpu.async_copy(table_h.at[si.at[0]], r0, gsem).wait()
                pltpu.sync_copy(r0, acc.at[di.at[0]], add=True)

        plsc.subcore_barrier()
        sl = acc.at[pl.ds(s * SLICE, SLICE)]

        @pl.when(c == 0)
        def _():
            pltpu.sync_copy(sl, out0_h.at[pl.ds(s * SLICE, SLICE)])

        @pl.when(c == 1)
        def _():
            pltpu.sync_copy(sl, out1_h.at[pl.ds(s * SLICE, SLICE)])

    return k(table, src2d, dst2d, zrows)


def _sc_gather_pair(task_tab, plat_tab, src2d, dst2d):
    nb = src2d.shape[0]
    n_full, n_rem = _split_blocks(nb)
    nfb = n_full // G_IDX
    e_out = nb * BLK
    mesh = plsc.VectorSubcoreMesh(core_axis_name="c", subcore_axis_name="s")

    @functools.partial(
        pl.kernel,
        out_type=(jax.ShapeDtypeStruct((e_out, EMB), jnp.bfloat16),
                  jax.ShapeDtypeStruct((e_out, EMB), jnp.bfloat16)),
        mesh=mesh,
        scratch_types=[
            pltpu.VMEM((G_IDX, BLK), jnp.int32),
            pltpu.VMEM((G_IDX, BLK), jnp.int32),
            pltpu.VMEM((BLK, EMB), jnp.bfloat16),
            pltpu.VMEM((BLK, EMB), jnp.bfloat16),
            pltpu.VMEM((BLK, EMB), jnp.bfloat16),
            pltpu.VMEM((BLK, EMB), jnp.bfloat16),
            pltpu.SemaphoreType.DMA,
            pltpu.SemaphoreType.DMA,
            pltpu.SemaphoreType.DMA,
        ],
        compiler_params=pltpu.CompilerParams(use_tc_tiling_on_sc=False),
    )
    def k(tt_h, pt_h, src_h, dst_h, gt_h, gp_h,
          si, di, rt0, rt1, rp0, rp1, isem, gsem, wsem):
        c = lax.axis_index("c")
        s = lax.axis_index("s")
        w = s * 2 + c
        base = w * n_full + jnp.minimum(w, n_rem)
        tbuf = (rt0, rt1)
        pbuf = (rp0, rp1)

        def run_batch(b0):
            cpa = pltpu.async_copy(src_h.at[pl.ds(b0, G_IDX)], si, isem)
            cpb = pltpu.async_copy(dst_h.at[pl.ds(b0, G_IDX)], di, isem)
            cpa.wait()
            cpb.wait()
            gat = [None] * G_IDX
            wrt = [None] * G_IDX
            gat[0] = (pltpu.async_copy(tt_h.at[si.at[0]], tbuf[0], gsem),
                      pltpu.async_copy(pt_h.at[di.at[0]], pbuf[0], gsem))
            for j in range(G_IDX):
                gat[j][0].wait()
                gat[j][1].wait()
                if j >= 1:
                    wrt[j - 1][0].wait()
                    wrt[j - 1][1].wait()
                if j + 1 < G_IDX:
                    nxt = (j + 1) % 2
                    gat[j + 1] = (
                        pltpu.async_copy(tt_h.at[si.at[j + 1]], tbuf[nxt], gsem),
                        pltpu.async_copy(pt_h.at[di.at[j + 1]], pbuf[nxt], gsem))
                off = (b0 + j) * BLK
                wrt[j] = (
                    pltpu.async_copy(tbuf[j % 2], gt_h.at[pl.ds(off, BLK)], wsem),
                    pltpu.async_copy(pbuf[j % 2], gp_h.at[pl.ds(off, BLK)], wsem))
            wrt[G_IDX - 1][0].wait()
            wrt[G_IDX - 1][1].wait()

        def body(i, carry):
            run_batch(base + i * G_IDX)
            return carry

        lax.fori_loop(0, nfb, body, 0)
        if n_rem:
            @pl.when(w < n_rem)
            def _():
                b = base + n_full
                pltpu.sync_copy(src_h.at[pl.ds(b, 1)], si.at[pl.ds(0, 1)])
                pltpu.sync_copy(dst_h.at[pl.ds(b, 1)], di.at[pl.ds(0, 1)])
                cp0 = pltpu.async_copy(tt_h.at[si.at[0]], rt0, gsem)
                cp1 = pltpu.async_copy(pt_h.at[di.at[0]], rp0, gsem)
                cp0.wait()
                cp1.wait()
                pltpu.sync_copy(rt0, gt_h.at[pl.ds(b * BLK, BLK)])
                pltpu.sync_copy(rp0, gp_h.at[pl.ds(b * BLK, BLK)])

    return k(task_tab, plat_tab, src2d, dst2d)


def _enc_body(x_r, w1_r, b1_r, g_r, be_r, w2_r, b2_r, o_r):
    h = jnp.dot(x_r[...], w1_r[...], preferred_element_type=_f32) + b1_r[...]
    m = jnp.mean(h, -1, keepdims=True)
    v = jnp.mean((h - m) * (h - m), -1, keepdims=True)
    h = (h - m) / jnp.sqrt(v + 1e-5) * g_r[...] + be_r[...]
    h = jnp.maximum(h, 0.0)
    o_r[...] = jnp.dot(h, w2_r[...], preferred_element_type=_f32) + b2_r[...]


def _encoder(x, w1, b1, g, be, w2, b2, blk=1000):
    n, f = x.shape
    grid = (n // blk,)
    full = lambda i: (0, 0)
    return pl.pallas_call(
        _enc_body,
        grid=grid,
        in_specs=[
            pl.BlockSpec((blk, f), lambda i: (i, 0)),
            pl.BlockSpec((f, HID), full),
            pl.BlockSpec((1, HID), full),
            pl.BlockSpec((1, HID), full),
            pl.BlockSpec((1, HID), full),
            pl.BlockSpec((HID, EMB), full),
            pl.BlockSpec((1, EMB), full),
        ],
        out_specs=pl.BlockSpec((blk, EMB), lambda i: (i, 0)),
        out_shape=jax.ShapeDtypeStruct((n, EMB), _f32),
    )(x, w1, b1, g, be, w2, b2)


def _task_mlp1(te, w1, b1, w2, b2, blk=1000):
    n = te.shape[0]

    def body(x_r, w1_r, b1_r, w2_r, b2_r, lo_r, hi_r):
        h = jnp.maximum(jnp.dot(x_r[...], w1_r[...], preferred_element_type=_f32)
                        + b1_r[...], 0.0)
        o = jnp.dot(h, w2_r[...], preferred_element_type=_f32) + b2_r[...]
        o = jnp.maximum(o, 0.0)
        lo_r[...] = o[:, :EMB]
        hi_r[...] = o[:, EMB:]

    full = lambda i: (0, 0)
    return pl.pallas_call(
        body,
        grid=(n // blk,),
        in_specs=[
            pl.BlockSpec((blk, EMB), lambda i: (i, 0)),
            pl.BlockSpec((EMB, HID), full),
            pl.BlockSpec((1, HID), full),
            pl.BlockSpec((HID, HID), full),
            pl.BlockSpec((1, HID), full),
        ],
        out_specs=(pl.BlockSpec((blk, EMB), lambda i: (i, 0)),
                   pl.BlockSpec((blk, EMB), lambda i: (i, 0))),
        out_shape=(jax.ShapeDtypeStruct((n, EMB), _f32),
                   jax.ShapeDtypeStruct((n, EMB), _f32)),
    )(te, w1, b1, w2, b2)


def _plat_mlp1(pe, a0, a1, w1, b1, w2, b2, blk=1000):
    n = pe.shape[0]

    def body(x_r, a0_r, a1_r, w1_r, b1_r, w2_r, b2_r, o_r):
        x = x_r[...] + a0_r[...] + a1_r[...]
        h = jnp.maximum(jnp.dot(x, w1_r[...], preferred_element_type=_f32)
                        + b1_r[...], 0.0)
        o = jnp.dot(h, w2_r[...], preferred_element_type=_f32) + b2_r[...]
        o_r[...] = jnp.maximum(o, 0.0)

    full = lambda i: (0, 0)
    blk_spec = pl.BlockSpec((blk, EMB), lambda i: (i, 0))
    return pl.pallas_call(
        body,
        grid=(n // blk,),
        in_specs=[
            blk_spec, blk_spec, blk_spec,
            pl.BlockSpec((EMB, HID), full),
            pl.BlockSpec((1, HID), full),
            pl.BlockSpec((HID, HID), full),
            pl.BlockSpec((1, HID), full),
        ],
        out_specs=pl.BlockSpec((blk, HID), lambda i: (i, 0)),
        out_shape=jax.ShapeDtypeStruct((n, HID), _f32),
    )(pe, a0, a1, w1, b1, w2, b2)


def _task_mlp23(t2lo, t2hi, g2_w1, g2_b1, g2_w2, g2_b2, g3_w1, g3_b1, g3_w2,
                g3_b2, blk=1000):
    n = t2lo.shape[0]

    def body(lo_r, hi_r, w1_r, b1_r, w2_r, b2_r, v1_r, c1_r, v2_r, c2_r,
             y3_r, t4_r):
        x = jnp.concatenate([lo_r[...], hi_r[...]], axis=-1)
        h = jnp.maximum(jnp.dot(x, w1_r[...], preferred_element_type=_f32)
                        + b1_r[...], 0.0)
        x3 = jnp.maximum(jnp.dot(h, w2_r[...], preferred_element_type=_f32)
                         + b2_r[...], 0.0)
        y3 = jnp.dot(x3, v1_r[...], preferred_element_type=_f32)
        y3_r[...] = y3
        h3 = jnp.maximum(y3 + c1_r[...], 0.0)
        t4 = jnp.dot(h3, v2_r[...], preferred_element_type=_f32) + c2_r[...]
        t4_r[...] = t4.astype(jnp.bfloat16)

    full = lambda i: (0, 0)
    rows64 = pl.BlockSpec((blk, EMB), lambda i: (i, 0))
    return pl.pallas_call(
        body,
        grid=(n // blk,),
        in_specs=[
            rows64, rows64,
            pl.BlockSpec((HID, HID), full),
            pl.BlockSpec((1, HID), full),
            pl.BlockSpec((HID, HID), full),
            pl.BlockSpec((1, HID), full),
            pl.BlockSpec((HID, EMB), full),
            pl.BlockSpec((1, EMB), full),
            pl.BlockSpec((EMB, EMB), full),
            pl.BlockSpec((1, EMB), full),
        ],
        out_specs=(rows64, rows64),
        out_shape=(jax.ShapeDtypeStruct((n, EMB), _f32),
                   jax.ShapeDtypeStruct((n, EMB), jnp.bfloat16)),
    )(t2lo, t2hi, g2_w1, g2_b1, g2_w2, g2_b2, g3_w1, g3_b1, g3_w2, g3_b2)


def _plat_mlp2(x2p, alo0, alo1, ahi0, ahi1, w1, b1, w2, b2, blk=1000):
    n = x2p.shape[0]

    def body(x_r, l0, l1, h0, h1, w1_r, b1_r, w2_r, b2_r, o_r):
        a = jnp.concatenate([l0[...] + l1[...], h0[...] + h1[...]], axis=-1)
        x = x_r[...] + a
        h = jnp.maximum(jnp.dot(x, w1_r[...], preferred_element_type=_f32)
                        + b1_r[...], 0.0)
        o = jnp.dot(h, w2_r[...], preferred_element_type=_f32) + b2_r[...]
        o_r[...] = jnp.maximum(o, 0.0)

    full = lambda i: (0, 0)
    rows64 = pl.BlockSpec((blk, EMB), lambda i: (i, 0))
    rows128 = pl.BlockSpec((blk, HID), lambda i: (i, 0))
    return pl.pallas_call(
        body,
        grid=(n // blk,),
        in_specs=[
            rows128, rows64, rows64, rows64, rows64,
            pl.BlockSpec((HID, HID), full),
            pl.BlockSpec((1, HID), full),
            pl.BlockSpec((HID, HID), full),
            pl.BlockSpec((1, HID), full),
        ],
        out_specs=rows128,
        out_shape=jax.ShapeDtypeStruct((n, HID), _f32),
    )(x2p, alo0, alo1, ahi0, ahi1, w1, b1, w2, b2)


def _plat_mlp3(x3p, a0, a1, w1, b1, w2, b2, blk=1000):
    n = x3p.shape[0]

    def body(x_r, a0_r, a1_r, w1_r, b1_r, w2_r, b2_r, o_r):
        h = jnp.dot(x_r[...], w1_r[...], preferred_element_type=_f32)
        h = jnp.maximum(h + b1_r[...] + a0_r[...] + a1_r[...], 0.0)
        o = jnp.dot(h, w2_r[...], preferred_element_type=_f32) + b2_r[...]
        o_r[...] = o.astype(jnp.bfloat16)

    full = lambda i: (0, 0)
    rows64 = pl.BlockSpec((blk, EMB), lambda i: (i, 0))
    return pl.pallas_call(
        body,
        grid=(n // blk,),
        in_specs=[
            pl.BlockSpec((blk, HID), lambda i: (i, 0)),
            rows64, rows64,
            pl.BlockSpec((HID, EMB), full),
            pl.BlockSpec((1, EMB), full),
            pl.BlockSpec((EMB, EMB), full),
            pl.BlockSpec((1, EMB), full),
        ],
        out_specs=rows64,
        out_shape=jax.ShapeDtypeStruct((n, EMB), jnp.bfloat16),
    )(x3p, a0, a1, w1, b1, w2, b2)


def _scorer(gt, gp, attr, w1t, w1p, w1e, b1, w2, b2, blk=8000):
    e = attr.shape[0]

    def body(t_r, p_r, a_r, wt_r, wp_r, we_r, b1_r, w2_r, b2_r, o_r):
        h = (jnp.dot(t_r[...], wt_r[...], preferred_element_type=_f32)
             + jnp.dot(p_r[...], wp_r[...], preferred_element_type=_f32)
             + jnp.dot(a_r[...].astype(jnp.bfloat16), we_r[...],
                       preferred_element_type=_f32)
             + b1_r[...])
        h = jnp.maximum(h, 0.0).astype(jnp.bfloat16)
        s = jnp.dot(h, w2_r[...], preferred_element_type=_f32) + b2_r[...]
        o_r[...] = jnp.clip(s, -50.0, 50.0)

    full = lambda i: (0, 0)
    return pl.pallas_call(
        body,
        grid=(e // blk,),
        in_specs=[
            pl.BlockSpec((blk, EMB), lambda i: (i, 0)),
            pl.BlockSpec((blk, EMB), lambda i: (i, 0)),
            pl.BlockSpec((blk, 3), lambda i: (i, 0)),
            pl.BlockSpec((EMB, HID), full),
            pl.BlockSpec((EMB, HID), full),
            pl.BlockSpec((3, HID), full),
            pl.BlockSpec((1, HID), full),
            pl.BlockSpec((HID, 1), full),
            pl.BlockSpec((1, 1), full),
        ],
        out_specs=pl.BlockSpec((blk, 1), lambda i: (i, 0)),
        out_shape=jax.ShapeDtypeStruct((e, 1), _f32),
    )(gt, gp, attr, w1t, w1p, w1e, b1, w2, b2)


def kernel(task_features, platform_features, edge_index, edge_attr, n_tasks,
           n_platforms,
           te_w1, te_b1, te_g, te_be, te_w2, te_b2,
           pe_w1, pe_b1, pe_g, pe_be, pe_w2, pe_b2,
           g1_w1, g1_b1, g1_w2, g1_b2,
           g2_w1, g2_b1, g2_w2, g2_b2,
           g3_w1, g3_b1, g3_w2, g3_b2,
           es_w1, es_b1, es_w2, es_b2):
    n_t = task_features.shape[0]
    n_p = platform_features.shape[0]
    e = edge_attr.shape[0]
    assert e % BLK == 0
    nb = e // BLK

    row = lambda v: v.reshape(1, -1).astype(_f32)
    delta = jnp.asarray((n_tasks + n_platforms) - (n_t + n_p)).astype(_f32)
    te_b2f = row(te_b2) + delta
    pe_b2f = row(pe_b2) + delta

    src = edge_index[0].astype(jnp.int32)
    dst_rel = (edge_index[1] - n_tasks).astype(jnp.int32)
    src2d = src.reshape(nb, BLK)
    dst2d = dst_rel.reshape(nb, BLK)
    zrows = jnp.zeros((SLICE, EMB), _f32)

    te = _encoder(task_features, te_w1, row(te_b1), row(te_g), row(te_be),
                  te_w2, te_b2f)
    pe = _encoder(platform_features, pe_w1, row(pe_b1), row(pe_g), row(pe_be),
                  pe_w2, pe_b2f)

    a1_0, a1_1 = _sc_segsum64(te, src2d, dst2d, zrows)
    t2lo, t2hi = _task_mlp1(te, g1_w1, row(g1_b1), g1_w2, row(g1_b2))
    x2p = _plat_mlp1(pe, a1_0, a1_1, g1_w1, row(g1_b1), g1_w2, row(g1_b2))

    a2lo0, a2lo1 = _sc_segsum64(t2lo, src2d, dst2d, zrows)
    a2hi0, a2hi1 = _sc_segsum64(t2hi, src2d, dst2d, zrows)
    y3, t4 = _task_mlp23(t2lo, t2hi, g2_w1, row(g2_b1), g2_w2, row(g2_b2),
                         g3_w1, row(g3_b1), g3_w2, row(g3_b2))
    x3p = _plat_mlp2(x2p, a2lo0, a2lo1, a2hi0, a2hi1,
                     g2_w1, row(g2_b1), g2_w2, row(g2_b2))

    a3_0, a3_1 = _sc_segsum64(y3, src2d, dst2d, zrows)
    p4 = _plat_mlp3(x3p, a3_0, a3_1, g3_w1, row(g3_b1), g3_w2, row(g3_b2))

    gt, gp = _sc_gather_pair(t4, p4, src2d, dst2d)
    s = _scorer(gt, gp, edge_attr,
                es_w1[:EMB], es_w1[EMB:2 * EMB], es_w1[2 * EMB:],
                row(es_b1), es_w2, jnp.reshape(es_b2, (1, 1)))
    return jnp.squeeze(s, -1)


# speedup vs baseline: 1.3920x; 1.3920x over previous
import functools

import jax
import jax.numpy as jnp
from jax import lax
from jax.experimental import pallas as pl
from jax.experimental.pallas import tpu as pltpu
from jax.experimental.pallas import tpu_sc as plsc

EMB = 64
HID = 128
NW = 32
BLK = 128
SLICE = 1568
PAD_P = 16 * SLICE

_f32 = jnp.float32


G_IDX = 15


def _split_blocks(nb):
    n_full = nb // NW
    n_rem = nb - n_full * NW
    assert n_full % G_IDX == 0
    return n_full, n_rem


def _sc_segsum64(table, src2d, dst2d, zrows):
    nb = src2d.shape[0]
    n_full, n_rem = _split_blocks(nb)
    nfb = n_full // G_IDX
    mesh = plsc.VectorSubcoreMesh(core_axis_name="c", subcore_axis_name="s")

    @functools.partial(
        pl.kernel,
        out_type=(jax.ShapeDtypeStruct((PAD_P, EMB), _f32),
                  jax.ShapeDtypeStruct((PAD_P, EMB), _f32)),
        mesh=mesh,
        scratch_types=[
            pltpu.VMEM_SHARED((PAD_P, EMB), _f32),
            pltpu.VMEM((G_IDX, BLK), jnp.int32),
            pltpu.VMEM((G_IDX, BLK), jnp.int32),
            pltpu.VMEM((BLK, EMB), _f32),
            pltpu.VMEM((BLK, EMB), _f32),
            pltpu.SemaphoreType.DMA,
            pltpu.SemaphoreType.DMA,
            pltpu.SemaphoreType.DMA,
        ],
        compiler_params=pltpu.CompilerParams(use_tc_tiling_on_sc=False),
    )
    def k(table_h, src_h, dst_h, z_h, out0_h, out1_h, acc,
          si, di, r0, r1, isem, gsem, ssem):
        c = lax.axis_index("c")
        s = lax.axis_index("s")
        w = s * 2 + c
        pltpu.sync_copy(z_h, acc.at[pl.ds(s * SLICE, SLICE)])
        plsc.subcore_barrier()
        base = w * n_full + jnp.minimum(w, n_rem)
        rbuf = (r0, r1)

        def run_batch(b0):
            cpa = pltpu.async_copy(src_h.at[pl.ds(b0, G_IDX)], si, isem)
            cpb = pltpu.async_copy(dst_h.at[pl.ds(b0, G_IDX)], di, isem)
            cpa.wait()
            cpb.wait()
            gat = [None] * G_IDX
            sca = [None] * G_IDX
            gat[0] = pltpu.async_copy(table_h.at[si.at[0]], rbuf[0], gsem)
            for j in range(G_IDX):
                gat[j].wait()
                if j >= 1:
                    sca[j - 1].wait()
                if j + 1 < G_IDX:
                    gat[j + 1] = pltpu.async_copy(
                        table_h.at[si.at[j + 1]], rbuf[(j + 1) % 2], gsem)
                sca[j] = pltpu.async_copy(
                    rbuf[j % 2], acc.at[di.at[j]], ssem, add=True)
            sca[G_IDX - 1].wait()

        def body(i, carry):
            run_batch(base + i * G_IDX)
            return carry

        lax.fori_loop(0, nfb, body, 0)
        if n_rem:
            @pl.when(w < n_rem)
            def _():
                b = base + n_full
                pltpu.sync_copy(src_h.at[pl.ds(b, 1)], si.at[pl.ds(0, 1)])
                pltpu.sync_copy(dst_h.at[pl.ds(b, 1)], di.at[pl.ds(0, 1)])
                pltpu.async_copy(table_h.at[si.at[0]], r0, gsem).wait()
                pltpu.sync_copy(r0, acc.at[di.at[0]], add=True)

        plsc.subcore_barrier()
        sl = acc.at[pl.ds(s * SLICE, SLICE)]

        @pl.when(c == 0)
        def _():
            pltpu.sync_copy(sl, out0_h.at[pl.ds(s * SLICE, SLICE)])

        @pl.when(c == 1)
        def _():
            pltpu.sync_copy(sl, out1_h.at[pl.ds(s * SLICE, SLICE)])

    return k(table, src2d, dst2d, zrows)


def _sc_gather_pair(task_tab, plat_tab, src2d, dst2d):
    nb = src2d.shape[0]
    n_full, n_rem = _split_blocks(nb)
    nfb = n_full // G_IDX
    e_out = nb * BLK
    mesh = plsc.VectorSubcoreMesh(core_axis_name="c", subcore_axis_name="s")

    @functools.partial(
        pl.kernel,
        out_type=jax.ShapeDtypeStruct((e_out, HID), _f32),
        mesh=mesh,
        scratch_types=[
            pltpu.VMEM((G_IDX, BLK), jnp.int32),
            pltpu.VMEM((G_IDX, BLK), jnp.int32),
            pltpu.VMEM((BLK, EMB), _f32),
            pltpu.VMEM((BLK, EMB), _f32),
            pltpu.VMEM((BLK, EMB), _f32),
            pltpu.VMEM((BLK, EMB), _f32),
            pltpu.SemaphoreType.DMA,
            pltpu.SemaphoreType.DMA,
            pltpu.SemaphoreType.DMA,
        ],
        compiler_params=pltpu.CompilerParams(use_tc_tiling_on_sc=False),
    )
    def k(tt_h, pt_h, src_h, dst_h, gc_h,
          si, di, rt0, rt1, rp0, rp1, isem, gsem, wsem):
        c = lax.axis_index("c")
        s = lax.axis_index("s")
        w = s * 2 + c
        base = w * n_full + jnp.minimum(w, n_rem)
        tbuf = (rt0, rt1)
        pbuf = (rp0, rp1)

        def run_batch(b0):
            cpa = pltpu.async_copy(src_h.at[pl.ds(b0, G_IDX)], si, isem)
            cpb = pltpu.async_copy(dst_h.at[pl.ds(b0, G_IDX)], di, isem)
            cpa.wait()
            cpb.wait()
            gat = [None] * G_IDX
            wrt = [None] * G_IDX
            gat[0] = (pltpu.async_copy(tt_h.at[si.at[0]], tbuf[0], gsem),
                      pltpu.async_copy(pt_h.at[di.at[0]], pbuf[0], gsem))
            for j in range(G_IDX):
                gat[j][0].wait()
                gat[j][1].wait()
                if j >= 1:
                    wrt[j - 1][0].wait()
                    wrt[j - 1][1].wait()
                if j + 1 < G_IDX:
                    nxt = (j + 1) % 2
                    gat[j + 1] = (
                        pltpu.async_copy(tt_h.at[si.at[j + 1]], tbuf[nxt], gsem),
                        pltpu.async_copy(pt_h.at[di.at[j + 1]], pbuf[nxt], gsem))
                off = (b0 + j) * BLK
                wrt[j] = (
                    pltpu.async_copy(
                        tbuf[j % 2],
                        gc_h.at[pl.ds(off, BLK), pl.ds(0, EMB)], wsem),
                    pltpu.async_copy(
                        pbuf[j % 2],
                        gc_h.at[pl.ds(off, BLK), pl.ds(EMB, EMB)], wsem))
            wrt[G_IDX - 1][0].wait()
            wrt[G_IDX - 1][1].wait()

        def body(i, carry):
            run_batch(base + i * G_IDX)
            return carry

        lax.fori_loop(0, nfb, body, 0)
        if n_rem:
            @pl.when(w < n_rem)
            def _():
                b = base + n_full
                pltpu.sync_copy(src_h.at[pl.ds(b, 1)], si.at[pl.ds(0, 1)])
                pltpu.sync_copy(dst_h.at[pl.ds(b, 1)], di.at[pl.ds(0, 1)])
                cp0 = pltpu.async_copy(tt_h.at[si.at[0]], rt0, gsem)
                cp1 = pltpu.async_copy(pt_h.at[di.at[0]], rp0, gsem)
                cp0.wait()
                cp1.wait()
                pltpu.sync_copy(rt0, gc_h.at[pl.ds(b * BLK, BLK), pl.ds(0, EMB)])
                pltpu.sync_copy(rp0, gc_h.at[pl.ds(b * BLK, BLK), pl.ds(EMB, EMB)])

    return k(task_tab, plat_tab, src2d, dst2d)


def _enc_body(x_r, w1_r, b1_r, g_r, be_r, w2_r, b2_r, o_r):
    h = jnp.dot(x_r[...], w1_r[...], preferred_element_type=_f32) + b1_r[...]
    m = jnp.mean(h, -1, keepdims=True)
    v = jnp.mean((h - m) * (h - m), -1, keepdims=True)
    h = (h - m) / jnp.sqrt(v + 1e-5) * g_r[...] + be_r[...]
    h = jnp.maximum(h, 0.0)
    o_r[...] = jnp.dot(h, w2_r[...], preferred_element_type=_f32) + b2_r[...]


def _encoder(x, w1, b1, g, be, w2, b2, blk=1000):
    n, f = x.shape
    grid = (n // blk,)
    full = lambda i: (0, 0)
    return pl.pallas_call(
        _enc_body,
        grid=grid,
        in_specs=[
            pl.BlockSpec((blk, f), lambda i: (i, 0)),
            pl.BlockSpec((f, HID), full),
            pl.BlockSpec((1, HID), full),
            pl.BlockSpec((1, HID), full),
            pl.BlockSpec((1, HID), full),
            pl.BlockSpec((HID, EMB), full),
            pl.BlockSpec((1, EMB), full),
        ],
        out_specs=pl.BlockSpec((blk, EMB), lambda i: (i, 0)),
        out_shape=jax.ShapeDtypeStruct((n, EMB), _f32),
    )(x, w1, b1, g, be, w2, b2)


def _task_mlp1(te, w1, b1, w2, b2, blk=1000):
    n = te.shape[0]

    def body(x_r, w1_r, b1_r, w2_r, b2_r, lo_r, hi_r):
        h = jnp.maximum(jnp.dot(x_r[...], w1_r[...], preferred_element_type=_f32)
                        + b1_r[...], 0.0)
        o = jnp.dot(h, w2_r[...], preferred_element_type=_f32) + b2_r[...]
        o = jnp.maximum(o, 0.0)
        lo_r[...] = o[:, :EMB]
        hi_r[...] = o[:, EMB:]

    full = lambda i: (0, 0)
    return pl.pallas_call(
        body,
        grid=(n // blk,),
        in_specs=[
            pl.BlockSpec((blk, EMB), lambda i: (i, 0)),
            pl.BlockSpec((EMB, HID), full),
            pl.BlockSpec((1, HID), full),
            pl.BlockSpec((HID, HID), full),
            pl.BlockSpec((1, HID), full),
        ],
        out_specs=(pl.BlockSpec((blk, EMB), lambda i: (i, 0)),
                   pl.BlockSpec((blk, EMB), lambda i: (i, 0))),
        out_shape=(jax.ShapeDtypeStruct((n, EMB), _f32),
                   jax.ShapeDtypeStruct((n, EMB), _f32)),
    )(te, w1, b1, w2, b2)


def _plat_mlp1(pe, a0, a1, w1, b1, w2, b2, blk=1000):
    n = pe.shape[0]

    def body(x_r, a0_r, a1_r, w1_r, b1_r, w2_r, b2_r, o_r):
        x = x_r[...] + a0_r[...] + a1_r[...]
        h = jnp.maximum(jnp.dot(x, w1_r[...], preferred_element_type=_f32)
                        + b1_r[...], 0.0)
        o = jnp.dot(h, w2_r[...], preferred_element_type=_f32) + b2_r[...]
        o_r[...] = jnp.maximum(o, 0.0)

    full = lambda i: (0, 0)
    blk_spec = pl.BlockSpec((blk, EMB), lambda i: (i, 0))
    return pl.pallas_call(
        body,
        grid=(n // blk,),
        in_specs=[
            blk_spec, blk_spec, blk_spec,
            pl.BlockSpec((EMB, HID), full),
            pl.BlockSpec((1, HID), full),
            pl.BlockSpec((HID, HID), full),
            pl.BlockSpec((1, HID), full),
        ],
        out_specs=pl.BlockSpec((blk, HID), lambda i: (i, 0)),
        out_shape=jax.ShapeDtypeStruct((n, HID), _f32),
    )(pe, a0, a1, w1, b1, w2, b2)


def _task_mlp23(t2lo, t2hi, g2_w1, g2_b1, g2_w2, g2_b2, g3_w1, g3_b1, g3_w2,
                g3_b2, blk=1000):
    n = t2lo.shape[0]

    def body(lo_r, hi_r, w1_r, b1_r, w2_r, b2_r, v1_r, c1_r, v2_r, c2_r,
             y3_r, t4_r):
        x = jnp.concatenate([lo_r[...], hi_r[...]], axis=-1)
        h = jnp.maximum(jnp.dot(x, w1_r[...], preferred_element_type=_f32)
                        + b1_r[...], 0.0)
        x3 = jnp.maximum(jnp.dot(h, w2_r[...], preferred_element_type=_f32)
                         + b2_r[...], 0.0)
        y3 = jnp.dot(x3, v1_r[...], preferred_element_type=_f32)
        y3_r[...] = y3
        h3 = jnp.maximum(y3 + c1_r[...], 0.0)
        t4_r[...] = jnp.dot(h3, v2_r[...], preferred_element_type=_f32) + c2_r[...]

    full = lambda i: (0, 0)
    rows64 = pl.BlockSpec((blk, EMB), lambda i: (i, 0))
    return pl.pallas_call(
        body,
        grid=(n // blk,),
        in_specs=[
            rows64, rows64,
            pl.BlockSpec((HID, HID), full),
            pl.BlockSpec((1, HID), full),
            pl.BlockSpec((HID, HID), full),
            pl.BlockSpec((1, HID), full),
            pl.BlockSpec((HID, EMB), full),
            pl.BlockSpec((1, EMB), full),
            pl.BlockSpec((EMB, EMB), full),
            pl.BlockSpec((1, EMB), full),
        ],
        out_specs=(rows64, rows64),
        out_shape=(jax.ShapeDtypeStruct((n, EMB), _f32),
                   jax.ShapeDtypeStruct((n, EMB), _f32)),
    )(t2lo, t2hi, g2_w1, g2_b1, g2_w2, g2_b2, g3_w1, g3_b1, g3_w2, g3_b2)


def _plat_mlp2(x2p, alo0, alo1, ahi0, ahi1, w1, b1, w2, b2, blk=1000):
    n = x2p.shape[0]

    def body(x_r, l0, l1, h0, h1, w1_r, b1_r, w2_r, b2_r, o_r):
        a = jnp.concatenate([l0[...] + l1[...], h0[...] + h1[...]], axis=-1)
        x = x_r[...] + a
        h = jnp.maximum(jnp.dot(x, w1_r[...], preferred_element_type=_f32)
                        + b1_r[...], 0.0)
        o = jnp.dot(h, w2_r[...], preferred_element_type=_f32) + b2_r[...]
        o_r[...] = jnp.maximum(o, 0.0)

    full = lambda i: (0, 0)
    rows64 = pl.BlockSpec((blk, EMB), lambda i: (i, 0))
    rows128 = pl.BlockSpec((blk, HID), lambda i: (i, 0))
    return pl.pallas_call(
        body,
        grid=(n // blk,),
        in_specs=[
            rows128, rows64, rows64, rows64, rows64,
            pl.BlockSpec((HID, HID), full),
            pl.BlockSpec((1, HID), full),
            pl.BlockSpec((HID, HID), full),
            pl.BlockSpec((1, HID), full),
        ],
        out_specs=rows128,
        out_shape=jax.ShapeDtypeStruct((n, HID), _f32),
    )(x2p, alo0, alo1, ahi0, ahi1, w1, b1, w2, b2)


def _plat_mlp3(x3p, a0, a1, w1, b1, w2, b2, blk=1000):
    n = x3p.shape[0]

    def body(x_r, a0_r, a1_r, w1_r, b1_r, w2_r, b2_r, o_r):
        h = jnp.dot(x_r[...], w1_r[...], preferred_element_type=_f32)
        h = jnp.maximum(h + b1_r[...] + a0_r[...] + a1_r[...], 0.0)
        o_r[...] = jnp.dot(h, w2_r[...], preferred_element_type=_f32) + b2_r[...]

    full = lambda i: (0, 0)
    rows64 = pl.BlockSpec((blk, EMB), lambda i: (i, 0))
    return pl.pallas_call(
        body,
        grid=(n // blk,),
        in_specs=[
            pl.BlockSpec((blk, HID), lambda i: (i, 0)),
            rows64, rows64,
            pl.BlockSpec((HID, EMB), full),
            pl.BlockSpec((1, EMB), full),
            pl.BlockSpec((EMB, EMB), full),
            pl.BlockSpec((1, EMB), full),
        ],
        out_specs=rows64,
        out_shape=jax.ShapeDtypeStruct((n, EMB), _f32),
    )(x3p, a0, a1, w1, b1, w2, b2)


def _scorer(gc, attr, w1tp, w1e, b1, w2, b2, blk=4000):
    e = attr.shape[0]

    def body(g_r, a_r, wtp_r, we_r, b1_r, w2_r, b2_r, o_r):
        h = (jnp.dot(g_r[...], wtp_r[...], preferred_element_type=_f32)
             + jnp.dot(a_r[...], we_r[...], preferred_element_type=_f32)
             + b1_r[...])
        h = jnp.maximum(h, 0.0)
        s = jnp.dot(h, w2_r[...], preferred_element_type=_f32) + b2_r[...]
        o_r[...] = jnp.clip(s, -50.0, 50.0)

    full = lambda i: (0, 0)
    return pl.pallas_call(
        body,
        grid=(e // blk,),
        in_specs=[
            pl.BlockSpec((blk, HID), lambda i: (i, 0)),
            pl.BlockSpec((blk, 3), lambda i: (i, 0)),
            pl.BlockSpec((HID, HID), full),
            pl.BlockSpec((3, HID), full),
            pl.BlockSpec((1, HID), full),
            pl.BlockSpec((HID, 1), full),
            pl.BlockSpec((1, 1), full),
        ],
        out_specs=pl.BlockSpec((blk, 1), lambda i: (i, 0)),
        out_shape=jax.ShapeDtypeStruct((e, 1), _f32),
    )(gc, attr, w1tp, w1e, b1, w2, b2)


def kernel(task_features, platform_features, edge_index, edge_attr, n_tasks,
           n_platforms,
           te_w1, te_b1, te_g, te_be, te_w2, te_b2,
           pe_w1, pe_b1, pe_g, pe_be, pe_w2, pe_b2,
           g1_w1, g1_b1, g1_w2, g1_b2,
           g2_w1, g2_b1, g2_w2, g2_b2,
           g3_w1, g3_b1, g3_w2, g3_b2,
           es_w1, es_b1, es_w2, es_b2):
    n_t = task_features.shape[0]
    n_p = platform_features.shape[0]
    e = edge_attr.shape[0]
    assert e % BLK == 0
    nb = e // BLK

    row = lambda v: v.reshape(1, -1).astype(_f32)
    delta = jnp.asarray((n_tasks + n_platforms) - (n_t + n_p)).astype(_f32)
    te_b2f = row(te_b2) + delta
    pe_b2f = row(pe_b2) + delta

    src = edge_index[0].astype(jnp.int32)
    dst_rel = (edge_index[1] - n_tasks).astype(jnp.int32)
    src2d = src.reshape(nb, BLK)
    dst2d = dst_rel.reshape(nb, BLK)
    zrows = jnp.zeros((SLICE, EMB), _f32)

    te = _encoder(task_features, te_w1, row(te_b1), row(te_g), row(te_be),
                  te_w2, te_b2f)
    pe = _encoder(platform_features, pe_w1, row(pe_b1), row(pe_g), row(pe_be),
                  pe_w2, pe_b2f)

    a1_0, a1_1 = _sc_segsum64(te, src2d, dst2d, zrows)
    t2lo, t2hi = _task_mlp1(te, g1_w1, row(g1_b1), g1_w2, row(g1_b2))
    x2p = _plat_mlp1(pe, a1_0, a1_1, g1_w1, row(g1_b1), g1_w2, row(g1_b2))

    a2lo0, a2lo1 = _sc_segsum64(t2lo, src2d, dst2d, zrows)
    a2hi0, a2hi1 = _sc_segsum64(t2hi, src2d, dst2d, zrows)
    y3, t4 = _task_mlp23(t2lo, t2hi, g2_w1, row(g2_b1), g2_w2, row(g2_b2),
                         g3_w1, row(g3_b1), g3_w2, row(g3_b2))
    x3p = _plat_mlp2(x2p, a2lo0, a2lo1, a2hi0, a2hi1,
                     g2_w1, row(g2_b1), g2_w2, row(g2_b2))

    a3_0, a3_1 = _sc_segsum64(y3, src2d, dst2d, zrows)
    p4 = _plat_mlp3(x3p, a3_0, a3_1, g3_w1, row(g3_b1), g3_w2, row(g3_b2))

    gc = _sc_gather_pair(t4, p4, src2d, dst2d)
    s = _scorer(gc, edge_attr,
                es_w1[:2 * EMB], es_w1[2 * EMB:],
                row(es_b1), es_w2, jnp.reshape(es_b2, (1, 1)))
    return jnp.squeeze(s, -1)


# speedup vs baseline: 1.6482x; 1.1840x over previous
import functools

import jax
import jax.numpy as jnp
from jax import lax
from jax.experimental import pallas as pl
from jax.experimental.pallas import tpu as pltpu
from jax.experimental.pallas import tpu_sc as plsc

EMB = 64
HID = 128
NW = 32
BLK = 128
SLICE = 1568
PAD_P = 16 * SLICE

_f32 = jnp.float32


G_IDX = 15


def _split_blocks(nb):
    n_full = nb // NW
    n_rem = nb - n_full * NW
    assert n_full % G_IDX == 0
    return n_full, n_rem


def _sc_segsum64(table, src2d, dst2d, zrows):
    nb = src2d.shape[0]
    n_full, n_rem = _split_blocks(nb)
    nfb = n_full // G_IDX
    mesh = plsc.VectorSubcoreMesh(core_axis_name="c", subcore_axis_name="s")

    @functools.partial(
        pl.kernel,
        out_type=(jax.ShapeDtypeStruct((PAD_P, EMB), _f32),
                  jax.ShapeDtypeStruct((PAD_P, EMB), _f32)),
        mesh=mesh,
        scratch_types=[
            pltpu.VMEM_SHARED((PAD_P, EMB), _f32),
            pltpu.VMEM((G_IDX, BLK), jnp.int32),
            pltpu.VMEM((G_IDX, BLK), jnp.int32),
            pltpu.VMEM((BLK, EMB), _f32),
            pltpu.VMEM((BLK, EMB), _f32),
            pltpu.VMEM((BLK, EMB), _f32),
            pltpu.SemaphoreType.DMA,
            pltpu.SemaphoreType.DMA,
            pltpu.SemaphoreType.DMA,
            pltpu.SemaphoreType.DMA,
            pltpu.SemaphoreType.DMA,
        ],
        compiler_params=pltpu.CompilerParams(use_tc_tiling_on_sc=False),
    )
    def k(table_h, src_h, dst_h, z_h, out0_h, out1_h, acc,
          si, di, r0, r1, r2, isem, gs0, gs1, gs2, ssem):
        c = lax.axis_index("c")
        s = lax.axis_index("s")
        w = s * 2 + c
        pltpu.sync_copy(z_h, acc.at[pl.ds(s * SLICE, SLICE)])
        plsc.subcore_barrier()
        base = w * n_full + jnp.minimum(w, n_rem)
        rbuf = (r0, r1, r2)
        gsem = (gs0, gs1, gs2)

        def run_batch(b0):
            cpa = pltpu.async_copy(src_h.at[pl.ds(b0, G_IDX)], si, isem)
            cpb = pltpu.async_copy(dst_h.at[pl.ds(b0, G_IDX)], di, isem)
            cpa.wait()
            cpb.wait()
            gat = [None] * G_IDX
            sca = [None] * G_IDX
            gat[0] = pltpu.async_copy(table_h.at[si.at[0]], rbuf[0], gsem[0])
            gat[1] = pltpu.async_copy(table_h.at[si.at[1]], rbuf[1], gsem[1])
            for j in range(G_IDX):
                gat[j].wait()
                if j >= 1:
                    sca[j - 1].wait()
                if j + 2 < G_IDX:
                    gat[j + 2] = pltpu.async_copy(
                        table_h.at[si.at[j + 2]], rbuf[(j + 2) % 3],
                        gsem[(j + 2) % 3])
                sca[j] = pltpu.async_copy(
                    rbuf[j % 3], acc.at[di.at[j]], ssem, add=True)
            sca[G_IDX - 1].wait()

        def body(i, carry):
            run_batch(base + i * G_IDX)
            return carry

        lax.fori_loop(0, nfb, body, 0)
        if n_rem:
            @pl.when(w < n_rem)
            def _():
                b = base + n_full
                pltpu.sync_copy(src_h.at[pl.ds(b, 1)], si.at[pl.ds(0, 1)])
                pltpu.sync_copy(dst_h.at[pl.ds(b, 1)], di.at[pl.ds(0, 1)])
                pltpu.async_copy(table_h.at[si.at[0]], r0, gs0).wait()
                pltpu.sync_copy(r0, acc.at[di.at[0]], add=True)

        plsc.subcore_barrier()
        sl = acc.at[pl.ds(s * SLICE, SLICE)]

        @pl.when(c == 0)
        def _():
            pltpu.sync_copy(sl, out0_h.at[pl.ds(s * SLICE, SLICE)])

        @pl.when(c == 1)
        def _():
            pltpu.sync_copy(sl, out1_h.at[pl.ds(s * SLICE, SLICE)])

    return k(table, src2d, dst2d, zrows)


def _sc_gather_pair(task_tab, plat_tab, src2d, dst2d):
    nb = src2d.shape[0]
    n_full, n_rem = _split_blocks(nb)
    nfb = n_full // G_IDX
    e_out = nb * BLK
    mesh = plsc.VectorSubcoreMesh(core_axis_name="c", subcore_axis_name="s")

    @functools.partial(
        pl.kernel,
        out_type=jax.ShapeDtypeStruct((e_out, HID), _f32),
        mesh=mesh,
        scratch_types=[
            pltpu.VMEM((G_IDX, BLK), jnp.int32),
            pltpu.VMEM((G_IDX, BLK), jnp.int32),
            pltpu.VMEM((BLK, EMB), _f32),
            pltpu.VMEM((BLK, EMB), _f32),
            pltpu.VMEM((BLK, EMB), _f32),
            pltpu.VMEM((BLK, EMB), _f32),
            pltpu.SemaphoreType.DMA,
            pltpu.SemaphoreType.DMA,
            pltpu.SemaphoreType.DMA,
        ],
        compiler_params=pltpu.CompilerParams(use_tc_tiling_on_sc=False),
    )
    def k(tt_h, pt_h, src_h, dst_h, gc_h,
          si, di, rt0, rt1, rp0, rp1, isem, gsem, wsem):
        c = lax.axis_index("c")
        s = lax.axis_index("s")
        w = s * 2 + c
        base = w * n_full + jnp.minimum(w, n_rem)
        tbuf = (rt0, rt1)
        pbuf = (rp0, rp1)

        def run_batch(b0):
            cpa = pltpu.async_copy(src_h.at[pl.ds(b0, G_IDX)], si, isem)
            cpb = pltpu.async_copy(dst_h.at[pl.ds(b0, G_IDX)], di, isem)
            cpa.wait()
            cpb.wait()
            gat = [None] * G_IDX
            wrt = [None] * G_IDX
            gat[0] = (pltpu.async_copy(tt_h.at[si.at[0]], tbuf[0], gsem),
                      pltpu.async_copy(pt_h.at[di.at[0]], pbuf[0], gsem))
            for j in range(G_IDX):
                gat[j][0].wait()
                gat[j][1].wait()
                if j >= 1:
                    wrt[j - 1][0].wait()
                    wrt[j - 1][1].wait()
                if j + 1 < G_IDX:
                    nxt = (j + 1) % 2
                    gat[j + 1] = (
                        pltpu.async_copy(tt_h.at[si.at[j + 1]], tbuf[nxt], gsem),
                        pltpu.async_copy(pt_h.at[di.at[j + 1]], pbuf[nxt], gsem))
                off = (b0 + j) * BLK
                wrt[j] = (
                    pltpu.async_copy(
                        tbuf[j % 2],
                        gc_h.at[pl.ds(off, BLK), pl.ds(0, EMB)], wsem),
                    pltpu.async_copy(
                        pbuf[j % 2],
                        gc_h.at[pl.ds(off, BLK), pl.ds(EMB, EMB)], wsem))
            wrt[G_IDX - 1][0].wait()
            wrt[G_IDX - 1][1].wait()

        def body(i, carry):
            run_batch(base + i * G_IDX)
            return carry

        lax.fori_loop(0, nfb, body, 0)
        if n_rem:
            @pl.when(w < n_rem)
            def _():
                b = base + n_full
                pltpu.sync_copy(src_h.at[pl.ds(b, 1)], si.at[pl.ds(0, 1)])
                pltpu.sync_copy(dst_h.at[pl.ds(b, 1)], di.at[pl.ds(0, 1)])
                cp0 = pltpu.async_copy(tt_h.at[si.at[0]], rt0, gsem)
                cp1 = pltpu.async_copy(pt_h.at[di.at[0]], rp0, gsem)
                cp0.wait()
                cp1.wait()
                pltpu.sync_copy(rt0, gc_h.at[pl.ds(b * BLK, BLK), pl.ds(0, EMB)])
                pltpu.sync_copy(rp0, gc_h.at[pl.ds(b * BLK, BLK), pl.ds(EMB, EMB)])

    return k(task_tab, plat_tab, src2d, dst2d)


def _enc_body(x_r, w1_r, b1_r, g_r, be_r, w2_r, b2_r, o_r):
    h = jnp.dot(x_r[...], w1_r[...], preferred_element_type=_f32) + b1_r[...]
    m = jnp.mean(h, -1, keepdims=True)
    v = jnp.mean((h - m) * (h - m), -1, keepdims=True)
    h = (h - m) / jnp.sqrt(v + 1e-5) * g_r[...] + be_r[...]
    h = jnp.maximum(h, 0.0)
    o_r[...] = jnp.dot(h, w2_r[...], preferred_element_type=_f32) + b2_r[...]


def _encoder(x, w1, b1, g, be, w2, b2, blk=1000):
    n, f = x.shape
    grid = (n // blk,)
    full = lambda i: (0, 0)
    return pl.pallas_call(
        _enc_body,
        grid=grid,
        in_specs=[
            pl.BlockSpec((blk, f), lambda i: (i, 0)),
            pl.BlockSpec((f, HID), full),
            pl.BlockSpec((1, HID), full),
            pl.BlockSpec((1, HID), full),
            pl.BlockSpec((1, HID), full),
            pl.BlockSpec((HID, EMB), full),
            pl.BlockSpec((1, EMB), full),
        ],
        out_specs=pl.BlockSpec((blk, EMB), lambda i: (i, 0)),
        out_shape=jax.ShapeDtypeStruct((n, EMB), _f32),
    )(x, w1, b1, g, be, w2, b2)


def _task_mlp1(te, w1, b1, w2, b2, blk=1000):
    n = te.shape[0]

    def body(x_r, w1_r, b1_r, w2_r, b2_r, lo_r, hi_r):
        h = jnp.maximum(jnp.dot(x_r[...], w1_r[...], preferred_element_type=_f32)
                        + b1_r[...], 0.0)
        o = jnp.dot(h, w2_r[...], preferred_element_type=_f32) + b2_r[...]
        o = jnp.maximum(o, 0.0)
        lo_r[...] = o[:, :EMB]
        hi_r[...] = o[:, EMB:]

    full = lambda i: (0, 0)
    return pl.pallas_call(
        body,
        grid=(n // blk,),
        in_specs=[
            pl.BlockSpec((blk, EMB), lambda i: (i, 0)),
            pl.BlockSpec((EMB, HID), full),
            pl.BlockSpec((1, HID), full),
            pl.BlockSpec((HID, HID), full),
            pl.BlockSpec((1, HID), full),
        ],
        out_specs=(pl.BlockSpec((blk, EMB), lambda i: (i, 0)),
                   pl.BlockSpec((blk, EMB), lambda i: (i, 0))),
        out_shape=(jax.ShapeDtypeStruct((n, EMB), _f32),
                   jax.ShapeDtypeStruct((n, EMB), _f32)),
    )(te, w1, b1, w2, b2)


def _plat_mlp1(pe, a0, a1, w1, b1, w2, b2, blk=1000):
    n = pe.shape[0]

    def body(x_r, a0_r, a1_r, w1_r, b1_r, w2_r, b2_r, o_r):
        x = x_r[...] + a0_r[...] + a1_r[...]
        h = jnp.maximum(jnp.dot(x, w1_r[...], preferred_element_type=_f32)
                        + b1_r[...], 0.0)
        o = jnp.dot(h, w2_r[...], preferred_element_type=_f32) + b2_r[...]
        o_r[...] = jnp.maximum(o, 0.0)

    full = lambda i: (0, 0)
    blk_spec = pl.BlockSpec((blk, EMB), lambda i: (i, 0))
    return pl.pallas_call(
        body,
        grid=(n // blk,),
        in_specs=[
            blk_spec, blk_spec, blk_spec,
            pl.BlockSpec((EMB, HID), full),
            pl.BlockSpec((1, HID), full),
            pl.BlockSpec((HID, HID), full),
            pl.BlockSpec((1, HID), full),
        ],
        out_specs=pl.BlockSpec((blk, HID), lambda i: (i, 0)),
        out_shape=jax.ShapeDtypeStruct((n, HID), _f32),
    )(pe, a0, a1, w1, b1, w2, b2)


def _task_mlp23(t2lo, t2hi, g2_w1, g2_b1, g2_w2, g2_b2, g3_w1, g3_b1, g3_w2,
                g3_b2, blk=1000):
    n = t2lo.shape[0]

    def body(lo_r, hi_r, w1_r, b1_r, w2_r, b2_r, v1_r, c1_r, v2_r, c2_r,
             y3_r, t4_r):
        x = jnp.concatenate([lo_r[...], hi_r[...]], axis=-1)
        h = jnp.maximum(jnp.dot(x, w1_r[...], preferred_element_type=_f32)
                        + b1_r[...], 0.0)
        x3 = jnp.maximum(jnp.dot(h, w2_r[...], preferred_element_type=_f32)
                         + b2_r[...], 0.0)
        y3 = jnp.dot(x3, v1_r[...], preferred_element_type=_f32)
        y3_r[...] = y3
        h3 = jnp.maximum(y3 + c1_r[...], 0.0)
        t4_r[...] = jnp.dot(h3, v2_r[...], preferred_element_type=_f32) + c2_r[...]

    full = lambda i: (0, 0)
    rows64 = pl.BlockSpec((blk, EMB), lambda i: (i, 0))
    return pl.pallas_call(
        body,
        grid=(n // blk,),
        in_specs=[
            rows64, rows64,
            pl.BlockSpec((HID, HID), full),
            pl.BlockSpec((1, HID), full),
            pl.BlockSpec((HID, HID), full),
            pl.BlockSpec((1, HID), full),
            pl.BlockSpec((HID, EMB), full),
            pl.BlockSpec((1, EMB), full),
            pl.BlockSpec((EMB, EMB), full),
            pl.BlockSpec((1, EMB), full),
        ],
        out_specs=(rows64, rows64),
        out_shape=(jax.ShapeDtypeStruct((n, EMB), _f32),
                   jax.ShapeDtypeStruct((n, EMB), _f32)),
    )(t2lo, t2hi, g2_w1, g2_b1, g2_w2, g2_b2, g3_w1, g3_b1, g3_w2, g3_b2)


def _plat_mlp2(x2p, alo0, alo1, ahi0, ahi1, w1, b1, w2, b2, blk=1000):
    n = x2p.shape[0]

    def body(x_r, l0, l1, h0, h1, w1_r, b1_r, w2_r, b2_r, o_r):
        a = jnp.concatenate([l0[...] + l1[...], h0[...] + h1[...]], axis=-1)
        x = x_r[...] + a
        h = jnp.maximum(jnp.dot(x, w1_r[...], preferred_element_type=_f32)
                        + b1_r[...], 0.0)
        o = jnp.dot(h, w2_r[...], preferred_element_type=_f32) + b2_r[...]
        o_r[...] = jnp.maximum(o, 0.0)

    full = lambda i: (0, 0)
    rows64 = pl.BlockSpec((blk, EMB), lambda i: (i, 0))
    rows128 = pl.BlockSpec((blk, HID), lambda i: (i, 0))
    return pl.pallas_call(
        body,
        grid=(n // blk,),
        in_specs=[
            rows128, rows64, rows64, rows64, rows64,
            pl.BlockSpec((HID, HID), full),
            pl.BlockSpec((1, HID), full),
            pl.BlockSpec((HID, HID), full),
            pl.BlockSpec((1, HID), full),
        ],
        out_specs=rows128,
        out_shape=jax.ShapeDtypeStruct((n, HID), _f32),
    )(x2p, alo0, alo1, ahi0, ahi1, w1, b1, w2, b2)


def _plat_mlp3(x3p, a0, a1, w1, b1, w2, b2, blk=1000):
    n = x3p.shape[0]

    def body(x_r, a0_r, a1_r, w1_r, b1_r, w2_r, b2_r, o_r):
        h = jnp.dot(x_r[...], w1_r[...], preferred_element_type=_f32)
        h = jnp.maximum(h + b1_r[...] + a0_r[...] + a1_r[...], 0.0)
        o_r[...] = jnp.dot(h, w2_r[...], preferred_element_type=_f32) + b2_r[...]

    full = lambda i: (0, 0)
    rows64 = pl.BlockSpec((blk, EMB), lambda i: (i, 0))
    return pl.pallas_call(
        body,
        grid=(n // blk,),
        in_specs=[
            pl.BlockSpec((blk, HID), lambda i: (i, 0)),
            rows64, rows64,
            pl.BlockSpec((HID, EMB), full),
            pl.BlockSpec((1, EMB), full),
            pl.BlockSpec((EMB, EMB), full),
            pl.BlockSpec((1, EMB), full),
        ],
        out_specs=rows64,
        out_shape=jax.ShapeDtypeStruct((n, EMB), _f32),
    )(x3p, a0, a1, w1, b1, w2, b2)


def _scorer(gc, attr, w1tp, w1e, b1, w2row, b2, blk=6400):
    e = attr.shape[0]

    def body(g_r, a_r, wtp_r, we_r, b1_r, w2_r, b2_r, o_r):
        h = (jnp.dot(g_r[...], wtp_r[...], preferred_element_type=_f32)
             + jnp.dot(a_r[...], we_r[...], preferred_element_type=_f32)
             + b1_r[...])
        h = jnp.maximum(h, 0.0)
        sv = jnp.sum(h * w2_r[...], axis=1) + jnp.reshape(b2_r[...], ())
        i = pl.program_id(0)
        o_r[pl.ds(i * blk, blk)] = jnp.clip(sv, -50.0, 50.0)

    full = lambda i: (0, 0)
    return pl.pallas_call(
        body,
        grid=(e // blk,),
        in_specs=[
            pl.BlockSpec((blk, HID), lambda i: (i, 0)),
            pl.BlockSpec((blk, 3), lambda i: (i, 0)),
            pl.BlockSpec((HID, HID), full),
            pl.BlockSpec((3, HID), full),
            pl.BlockSpec((1, HID), full),
            pl.BlockSpec((1, HID), full),
            pl.BlockSpec((1, 1), full),
        ],
        out_specs=pl.BlockSpec((e,), lambda i: (0,)),
        out_shape=jax.ShapeDtypeStruct((e,), _f32),
    )(gc, attr, w1tp, w1e, b1, w2row, b2)


def kernel(task_features, platform_features, edge_index, edge_attr, n_tasks,
           n_platforms,
           te_w1, te_b1, te_g, te_be, te_w2, te_b2,
           pe_w1, pe_b1, pe_g, pe_be, pe_w2, pe_b2,
           g1_w1, g1_b1, g1_w2, g1_b2,
           g2_w1, g2_b1, g2_w2, g2_b2,
           g3_w1, g3_b1, g3_w2, g3_b2,
           es_w1, es_b1, es_w2, es_b2):
    n_t = task_features.shape[0]
    n_p = platform_features.shape[0]
    e = edge_attr.shape[0]
    assert e % BLK == 0
    nb = e // BLK

    row = lambda v: v.reshape(1, -1).astype(_f32)
    delta = jnp.asarray((n_tasks + n_platforms) - (n_t + n_p)).astype(_f32)
    te_b2f = row(te_b2) + delta
    pe_b2f = row(pe_b2) + delta

    src = edge_index[0].astype(jnp.int32)
    dst_rel = (edge_index[1] - n_tasks).astype(jnp.int32)
    src2d = src.reshape(nb, BLK)
    dst2d = dst_rel.reshape(nb, BLK)
    zrows = jnp.zeros((SLICE, EMB), _f32)

    te = _encoder(task_features, te_w1, row(te_b1), row(te_g), row(te_be),
                  te_w2, te_b2f)
    pe = _encoder(platform_features, pe_w1, row(pe_b1), row(pe_g), row(pe_be),
                  pe_w2, pe_b2f)

    a1_0, a1_1 = _sc_segsum64(te, src2d, dst2d, zrows)
    t2lo, t2hi = _task_mlp1(te, g1_w1, row(g1_b1), g1_w2, row(g1_b2))
    x2p = _plat_mlp1(pe, a1_0, a1_1, g1_w1, row(g1_b1), g1_w2, row(g1_b2))

    a2lo0, a2lo1 = _sc_segsum64(t2lo, src2d, dst2d, zrows)
    a2hi0, a2hi1 = _sc_segsum64(t2hi, src2d, dst2d, zrows)
    y3, t4 = _task_mlp23(t2lo, t2hi, g2_w1, row(g2_b1), g2_w2, row(g2_b2),
                         g3_w1, row(g3_b1), g3_w2, row(g3_b2))
    x3p = _plat_mlp2(x2p, a2lo0, a2lo1, a2hi0, a2hi1,
                     g2_w1, row(g2_b1), g2_w2, row(g2_b2))

    a3_0, a3_1 = _sc_segsum64(y3, src2d, dst2d, zrows)
    p4 = _plat_mlp3(x3p, a3_0, a3_1, g3_w1, row(g3_b1), g3_w2, row(g3_b2))

    gc = _sc_gather_pair(t4, p4, src2d, dst2d)
    return _scorer(gc, edge_attr,
                   es_w1[:2 * EMB], es_w1[2 * EMB:],
                   row(es_b1), jnp.reshape(es_w2, (1, HID)),
                   jnp.reshape(es_b2, (1, 1)))


# speedup vs baseline: 1.8694x; 1.1342x over previous
import functools

import jax
import jax.numpy as jnp
from jax import lax
from jax.experimental import pallas as pl
from jax.experimental.pallas import tpu as pltpu
from jax.experimental.pallas import tpu_sc as plsc

EMB = 64
HID = 128
NW = 32
BLK = 128
SLICE = 1568
PAD_P = 16 * SLICE

_f32 = jnp.float32


G_IDX = 15


def _split_blocks(nb):
    n_full = nb // NW
    n_rem = nb - n_full * NW
    assert n_full % G_IDX == 0
    return n_full, n_rem


def _sc_segsum64(table, src2d, dst2d, zrows):
    nb = src2d.shape[0]
    n_full, n_rem = _split_blocks(nb)
    nfb = n_full // G_IDX
    mesh = plsc.VectorSubcoreMesh(core_axis_name="c", subcore_axis_name="s")

    @functools.partial(
        pl.kernel,
        out_type=(jax.ShapeDtypeStruct((PAD_P, EMB), _f32),
                  jax.ShapeDtypeStruct((PAD_P, EMB), _f32)),
        mesh=mesh,
        scratch_types=[
            pltpu.VMEM_SHARED((PAD_P, EMB), _f32),
            pltpu.VMEM((G_IDX, BLK), jnp.int32),
            pltpu.VMEM((G_IDX, BLK), jnp.int32),
            pltpu.VMEM((BLK, EMB), _f32),
            pltpu.VMEM((BLK, EMB), _f32),
            pltpu.VMEM((BLK, EMB), _f32),
            pltpu.SemaphoreType.DMA,
            pltpu.SemaphoreType.DMA,
            pltpu.SemaphoreType.DMA,
            pltpu.SemaphoreType.DMA,
            pltpu.SemaphoreType.DMA,
        ],
        compiler_params=pltpu.CompilerParams(use_tc_tiling_on_sc=False),
    )
    def k(table_h, src_h, dst_h, z_h, out0_h, out1_h, acc,
          si, di, r0, r1, r2, isem, gs0, gs1, gs2, ssem):
        c = lax.axis_index("c")
        s = lax.axis_index("s")
        w = s * 2 + c
        pltpu.sync_copy(z_h, acc.at[pl.ds(s * SLICE, SLICE)])
        plsc.subcore_barrier()
        base = w * n_full + jnp.minimum(w, n_rem)
        rbuf = (r0, r1, r2)
        gsem = (gs0, gs1, gs2)

        def run_batch(b0):
            cpa = pltpu.async_copy(src_h.at[pl.ds(b0, G_IDX)], si, isem)
            cpb = pltpu.async_copy(dst_h.at[pl.ds(b0, G_IDX)], di, isem)
            cpa.wait()
            cpb.wait()
            gat = [None] * G_IDX
            sca = [None] * G_IDX
            gat[0] = pltpu.async_copy(table_h.at[si.at[0]], rbuf[0], gsem[0])
            gat[1] = pltpu.async_copy(table_h.at[si.at[1]], rbuf[1], gsem[1])
            for j in range(G_IDX):
                gat[j].wait()
                if j >= 1:
                    sca[j - 1].wait()
                if j + 2 < G_IDX:
                    gat[j + 2] = pltpu.async_copy(
                        table_h.at[si.at[j + 2]], rbuf[(j + 2) % 3],
                        gsem[(j + 2) % 3])
                sca[j] = pltpu.async_copy(
                    rbuf[j % 3], acc.at[di.at[j]], ssem, add=True)
            sca[G_IDX - 1].wait()

        def body(i, carry):
            run_batch(base + i * G_IDX)
            return carry

        lax.fori_loop(0, nfb, body, 0)
        if n_rem:
            @pl.when(w < n_rem)
            def _():
                b = base + n_full
                pltpu.sync_copy(src_h.at[pl.ds(b, 1)], si.at[pl.ds(0, 1)])
                pltpu.sync_copy(dst_h.at[pl.ds(b, 1)], di.at[pl.ds(0, 1)])
                pltpu.async_copy(table_h.at[si.at[0]], r0, gs0).wait()
                pltpu.sync_copy(r0, acc.at[di.at[0]], add=True)

        plsc.subcore_barrier()
        sl = acc.at[pl.ds(s * SLICE, SLICE)]

        @pl.when(c == 0)
        def _():
            pltpu.sync_copy(sl, out0_h.at[pl.ds(s * SLICE, SLICE)])

        @pl.when(c == 1)
        def _():
            pltpu.sync_copy(sl, out1_h.at[pl.ds(s * SLICE, SLICE)])

    return k(table, src2d, dst2d, zrows)


def _sc_gather_pair(task_tab, plat_tab, src2d, dst2d):
    nb = src2d.shape[0]
    n_full, n_rem = _split_blocks(nb)
    nfb = n_full // G_IDX
    e_out = nb * BLK
    mesh = plsc.VectorSubcoreMesh(core_axis_name="c", subcore_axis_name="s")

    @functools.partial(
        pl.kernel,
        out_type=jax.ShapeDtypeStruct((e_out, HID), _f32),
        mesh=mesh,
        scratch_types=[
            pltpu.VMEM((G_IDX, BLK), jnp.int32),
            pltpu.VMEM((G_IDX, BLK), jnp.int32),
            pltpu.VMEM((BLK, EMB), _f32),
            pltpu.VMEM((BLK, EMB), _f32),
            pltpu.VMEM((BLK, EMB), _f32),
            pltpu.VMEM((BLK, EMB), _f32),
            pltpu.VMEM((BLK, EMB), _f32),
            pltpu.VMEM((BLK, EMB), _f32),
            pltpu.SemaphoreType.DMA,
            pltpu.SemaphoreType.DMA,
            pltpu.SemaphoreType.DMA,
            pltpu.SemaphoreType.DMA,
            pltpu.SemaphoreType.DMA,
        ],
        compiler_params=pltpu.CompilerParams(use_tc_tiling_on_sc=False),
    )
    def k(tt_h, pt_h, src_h, dst_h, gc_h,
          si, di, rt0, rt1, rt2, rp0, rp1, rp2, isem, gs0, gs1, gs2, wsem):
        c = lax.axis_index("c")
        s = lax.axis_index("s")
        w = s * 2 + c
        base = w * n_full + jnp.minimum(w, n_rem)
        tbuf = (rt0, rt1, rt2)
        pbuf = (rp0, rp1, rp2)
        gsem = (gs0, gs1, gs2)

        def run_batch(b0):
            cpa = pltpu.async_copy(src_h.at[pl.ds(b0, G_IDX)], si, isem)
            cpb = pltpu.async_copy(dst_h.at[pl.ds(b0, G_IDX)], di, isem)
            cpa.wait()
            cpb.wait()
            gat = [None] * G_IDX
            wrt = [None] * G_IDX

            def start_gat(j):
                k = j % 3
                return (pltpu.async_copy(tt_h.at[si.at[j]], tbuf[k], gsem[k]),
                        pltpu.async_copy(pt_h.at[di.at[j]], pbuf[k], gsem[k]))

            gat[0] = start_gat(0)
            gat[1] = start_gat(1)
            for j in range(G_IDX):
                gat[j][0].wait()
                gat[j][1].wait()
                if j >= 1:
                    wrt[j - 1][0].wait()
                    wrt[j - 1][1].wait()
                if j + 2 < G_IDX:
                    gat[j + 2] = start_gat(j + 2)
                off = (b0 + j) * BLK
                wrt[j] = (
                    pltpu.async_copy(
                        tbuf[j % 3],
                        gc_h.at[pl.ds(off, BLK), pl.ds(0, EMB)], wsem),
                    pltpu.async_copy(
                        pbuf[j % 3],
                        gc_h.at[pl.ds(off, BLK), pl.ds(EMB, EMB)], wsem))
            wrt[G_IDX - 1][0].wait()
            wrt[G_IDX - 1][1].wait()

        def body(i, carry):
            run_batch(base + i * G_IDX)
            return carry

        lax.fori_loop(0, nfb, body, 0)
        if n_rem:
            @pl.when(w < n_rem)
            def _():
                b = base + n_full
                pltpu.sync_copy(src_h.at[pl.ds(b, 1)], si.at[pl.ds(0, 1)])
                pltpu.sync_copy(dst_h.at[pl.ds(b, 1)], di.at[pl.ds(0, 1)])
                cp0 = pltpu.async_copy(tt_h.at[si.at[0]], rt0, gs0)
                cp1 = pltpu.async_copy(pt_h.at[di.at[0]], rp0, gs0)
                cp0.wait()
                cp1.wait()
                pltpu.sync_copy(rt0, gc_h.at[pl.ds(b * BLK, BLK), pl.ds(0, EMB)])
                pltpu.sync_copy(rp0, gc_h.at[pl.ds(b * BLK, BLK), pl.ds(EMB, EMB)])

    return k(task_tab, plat_tab, src2d, dst2d)


def _enc_body(x_r, w1_r, b1_r, g_r, be_r, w2_r, b2_r, o_r):
    h = jnp.dot(x_r[...], w1_r[...], preferred_element_type=_f32) + b1_r[...]
    m = jnp.mean(h, -1, keepdims=True)
    v = jnp.mean((h - m) * (h - m), -1, keepdims=True)
    h = (h - m) / jnp.sqrt(v + 1e-5) * g_r[...] + be_r[...]
    h = jnp.maximum(h, 0.0)
    o_r[...] = jnp.dot(h, w2_r[...], preferred_element_type=_f32) + b2_r[...]


def _encoder(x, w1, b1, g, be, w2, b2, blk=1000):
    n, f = x.shape
    grid = (n // blk,)
    full = lambda i: (0, 0)
    return pl.pallas_call(
        _enc_body,
        grid=grid,
        in_specs=[
            pl.BlockSpec((blk, f), lambda i: (i, 0)),
            pl.BlockSpec((f, HID), full),
            pl.BlockSpec((1, HID), full),
            pl.BlockSpec((1, HID), full),
            pl.BlockSpec((1, HID), full),
            pl.BlockSpec((HID, EMB), full),
            pl.BlockSpec((1, EMB), full),
        ],
        out_specs=pl.BlockSpec((blk, EMB), lambda i: (i, 0)),
        out_shape=jax.ShapeDtypeStruct((n, EMB), _f32),
    )(x, w1, b1, g, be, w2, b2)


def _task_mlp1(te, w1, b1, w2, b2, blk=1000):
    n = te.shape[0]

    def body(x_r, w1_r, b1_r, w2_r, b2_r, lo_r, hi_r):
        h = jnp.maximum(jnp.dot(x_r[...], w1_r[...], preferred_element_type=_f32)
                        + b1_r[...], 0.0)
        o = jnp.dot(h, w2_r[...], preferred_element_type=_f32) + b2_r[...]
        o = jnp.maximum(o, 0.0)
        lo_r[...] = o[:, :EMB]
        hi_r[...] = o[:, EMB:]

    full = lambda i: (0, 0)
    return pl.pallas_call(
        body,
        grid=(n // blk,),
        in_specs=[
            pl.BlockSpec((blk, EMB), lambda i: (i, 0)),
            pl.BlockSpec((EMB, HID), full),
            pl.BlockSpec((1, HID), full),
            pl.BlockSpec((HID, HID), full),
            pl.BlockSpec((1, HID), full),
        ],
        out_specs=(pl.BlockSpec((blk, EMB), lambda i: (i, 0)),
                   pl.BlockSpec((blk, EMB), lambda i: (i, 0))),
        out_shape=(jax.ShapeDtypeStruct((n, EMB), _f32),
                   jax.ShapeDtypeStruct((n, EMB), _f32)),
    )(te, w1, b1, w2, b2)


def _plat_mlp1(pe, a0, a1, w1, b1, w2, b2, blk=1000):
    n = pe.shape[0]

    def body(x_r, a0_r, a1_r, w1_r, b1_r, w2_r, b2_r, o_r):
        x = x_r[...] + a0_r[...] + a1_r[...]
        h = jnp.maximum(jnp.dot(x, w1_r[...], preferred_element_type=_f32)
                        + b1_r[...], 0.0)
        o = jnp.dot(h, w2_r[...], preferred_element_type=_f32) + b2_r[...]
        o_r[...] = jnp.maximum(o, 0.0)

    full = lambda i: (0, 0)
    blk_spec = pl.BlockSpec((blk, EMB), lambda i: (i, 0))
    return pl.pallas_call(
        body,
        grid=(n // blk,),
        in_specs=[
            blk_spec, blk_spec, blk_spec,
            pl.BlockSpec((EMB, HID), full),
            pl.BlockSpec((1, HID), full),
            pl.BlockSpec((HID, HID), full),
            pl.BlockSpec((1, HID), full),
        ],
        out_specs=pl.BlockSpec((blk, HID), lambda i: (i, 0)),
        out_shape=jax.ShapeDtypeStruct((n, HID), _f32),
    )(pe, a0, a1, w1, b1, w2, b2)


def _task_mlp23(t2lo, t2hi, g2_w1, g2_b1, g2_w2, g2_b2, g3_w1, g3_b1, g3_w2,
                g3_b2, blk=1000):
    n = t2lo.shape[0]

    def body(lo_r, hi_r, w1_r, b1_r, w2_r, b2_r, v1_r, c1_r, v2_r, c2_r,
             y3_r, t4_r):
        x = jnp.concatenate([lo_r[...], hi_r[...]], axis=-1)
        h = jnp.maximum(jnp.dot(x, w1_r[...], preferred_element_type=_f32)
                        + b1_r[...], 0.0)
        x3 = jnp.maximum(jnp.dot(h, w2_r[...], preferred_element_type=_f32)
                         + b2_r[...], 0.0)
        y3 = jnp.dot(x3, v1_r[...], preferred_element_type=_f32)
        y3_r[...] = y3
        h3 = jnp.maximum(y3 + c1_r[...], 0.0)
        t4_r[...] = jnp.dot(h3, v2_r[...], preferred_element_type=_f32) + c2_r[...]

    full = lambda i: (0, 0)
    rows64 = pl.BlockSpec((blk, EMB), lambda i: (i, 0))
    return pl.pallas_call(
        body,
        grid=(n // blk,),
        in_specs=[
            rows64, rows64,
            pl.BlockSpec((HID, HID), full),
            pl.BlockSpec((1, HID), full),
            pl.BlockSpec((HID, HID), full),
            pl.BlockSpec((1, HID), full),
            pl.BlockSpec((HID, EMB), full),
            pl.BlockSpec((1, EMB), full),
            pl.BlockSpec((EMB, EMB), full),
            pl.BlockSpec((1, EMB), full),
        ],
        out_specs=(rows64, rows64),
        out_shape=(jax.ShapeDtypeStruct((n, EMB), _f32),
                   jax.ShapeDtypeStruct((n, EMB), _f32)),
    )(t2lo, t2hi, g2_w1, g2_b1, g2_w2, g2_b2, g3_w1, g3_b1, g3_w2, g3_b2)


def _plat_mlp2(x2p, alo0, alo1, ahi0, ahi1, w1, b1, w2, b2, blk=1000):
    n = x2p.shape[0]

    def body(x_r, l0, l1, h0, h1, w1_r, b1_r, w2_r, b2_r, o_r):
        a = jnp.concatenate([l0[...] + l1[...], h0[...] + h1[...]], axis=-1)
        x = x_r[...] + a
        h = jnp.maximum(jnp.dot(x, w1_r[...], preferred_element_type=_f32)
                        + b1_r[...], 0.0)
        o = jnp.dot(h, w2_r[...], preferred_element_type=_f32) + b2_r[...]
        o_r[...] = jnp.maximum(o, 0.0)

    full = lambda i: (0, 0)
    rows64 = pl.BlockSpec((blk, EMB), lambda i: (i, 0))
    rows128 = pl.BlockSpec((blk, HID), lambda i: (i, 0))
    return pl.pallas_call(
        body,
        grid=(n // blk,),
        in_specs=[
            rows128, rows64, rows64, rows64, rows64,
            pl.BlockSpec((HID, HID), full),
            pl.BlockSpec((1, HID), full),
            pl.BlockSpec((HID, HID), full),
            pl.BlockSpec((1, HID), full),
        ],
        out_specs=rows128,
        out_shape=jax.ShapeDtypeStruct((n, HID), _f32),
    )(x2p, alo0, alo1, ahi0, ahi1, w1, b1, w2, b2)


def _plat_mlp3(x3p, a0, a1, w1, b1, w2, b2, blk=1000):
    n = x3p.shape[0]

    def body(x_r, a0_r, a1_r, w1_r, b1_r, w2_r, b2_r, o_r):
        h = jnp.dot(x_r[...], w1_r[...], preferred_element_type=_f32)
        h = jnp.maximum(h + b1_r[...] + a0_r[...] + a1_r[...], 0.0)
        o_r[...] = jnp.dot(h, w2_r[...], preferred_element_type=_f32) + b2_r[...]

    full = lambda i: (0, 0)
    rows64 = pl.BlockSpec((blk, EMB), lambda i: (i, 0))
    return pl.pallas_call(
        body,
        grid=(n // blk,),
        in_specs=[
            pl.BlockSpec((blk, HID), lambda i: (i, 0)),
            rows64, rows64,
            pl.BlockSpec((HID, EMB), full),
            pl.BlockSpec((1, EMB), full),
            pl.BlockSpec((EMB, EMB), full),
            pl.BlockSpec((1, EMB), full),
        ],
        out_specs=rows64,
        out_shape=jax.ShapeDtypeStruct((n, EMB), _f32),
    )(x3p, a0, a1, w1, b1, w2, b2)


def _scorer(gc, attr_t, w1tp, w1e, b1, w2row, b2, blk=6400):
    e = attr_t.shape[1]

    def body(g_r, a_r, wtp_r, we_r, b1_r, w2_r, b2_r, o_r):
        ue = lax.dot_general(a_r[...], we_r[...],
                             dimension_numbers=(((0,), (0,)), ((), ())),
                             preferred_element_type=_f32)
        h = (jnp.dot(g_r[...], wtp_r[...], preferred_element_type=_f32)
             + ue + b1_r[...])
        h = jnp.maximum(h, 0.0)
        sv = jnp.sum(h * w2_r[...], axis=1) + jnp.reshape(b2_r[...], ())
        i = pl.program_id(0)
        o_r[pl.ds(i * blk, blk)] = jnp.clip(sv, -50.0, 50.0)

    full = lambda i: (0, 0)
    return pl.pallas_call(
        body,
        grid=(e // blk,),
        in_specs=[
            pl.BlockSpec((blk, HID), lambda i: (i, 0)),
            pl.BlockSpec((3, blk), lambda i: (0, i)),
            pl.BlockSpec((HID, HID), full),
            pl.BlockSpec((3, HID), full),
            pl.BlockSpec((1, HID), full),
            pl.BlockSpec((1, HID), full),
            pl.BlockSpec((1, 1), full),
        ],
        out_specs=pl.BlockSpec((e,), lambda i: (0,)),
        out_shape=jax.ShapeDtypeStruct((e,), _f32),
    )(gc, attr_t, w1tp, w1e, b1, w2row, b2)


def kernel(task_features, platform_features, edge_index, edge_attr, n_tasks,
           n_platforms,
           te_w1, te_b1, te_g, te_be, te_w2, te_b2,
           pe_w1, pe_b1, pe_g, pe_be, pe_w2, pe_b2,
           g1_w1, g1_b1, g1_w2, g1_b2,
           g2_w1, g2_b1, g2_w2, g2_b2,
           g3_w1, g3_b1, g3_w2, g3_b2,
           es_w1, es_b1, es_w2, es_b2):
    n_t = task_features.shape[0]
    n_p = platform_features.shape[0]
    e = edge_attr.shape[0]
    assert e % BLK == 0
    nb = e // BLK

    row = lambda v: v.reshape(1, -1).astype(_f32)
    delta = jnp.asarray((n_tasks + n_platforms) - (n_t + n_p)).astype(_f32)
    te_b2f = row(te_b2) + delta
    pe_b2f = row(pe_b2) + delta

    src = edge_index[0].astype(jnp.int32)
    dst_rel = (edge_index[1] - n_tasks).astype(jnp.int32)
    src2d = src.reshape(nb, BLK)
    dst2d = dst_rel.reshape(nb, BLK)
    zrows = jnp.zeros((SLICE, EMB), _f32)

    te = _encoder(task_features, te_w1, row(te_b1), row(te_g), row(te_be),
                  te_w2, te_b2f)
    pe = _encoder(platform_features, pe_w1, row(pe_b1), row(pe_g), row(pe_be),
                  pe_w2, pe_b2f)

    a1_0, a1_1 = _sc_segsum64(te, src2d, dst2d, zrows)
    t2lo, t2hi = _task_mlp1(te, g1_w1, row(g1_b1), g1_w2, row(g1_b2))
    x2p = _plat_mlp1(pe, a1_0, a1_1, g1_w1, row(g1_b1), g1_w2, row(g1_b2))

    a2lo0, a2lo1 = _sc_segsum64(t2lo, src2d, dst2d, zrows)
    a2hi0, a2hi1 = _sc_segsum64(t2hi, src2d, dst2d, zrows)
    y3, t4 = _task_mlp23(t2lo, t2hi, g2_w1, row(g2_b1), g2_w2, row(g2_b2),
                         g3_w1, row(g3_b1), g3_w2, row(g3_b2))
    x3p = _plat_mlp2(x2p, a2lo0, a2lo1, a2hi0, a2hi1,
                     g2_w1, row(g2_b1), g2_w2, row(g2_b2))

    a3_0, a3_1 = _sc_segsum64(y3, src2d, dst2d, zrows)
    p4 = _plat_mlp3(x3p, a3_0, a3_1, g3_w1, row(g3_b1), g3_w2, row(g3_b2))

    gc = _sc_gather_pair(t4, p4, src2d, dst2d)
    return _scorer(gc, edge_attr.T,
                   es_w1[:2 * EMB], es_w1[2 * EMB:],
                   row(es_b1), jnp.reshape(es_w2, (1, HID)),
                   jnp.reshape(es_b2, (1, 1)))
